# baseline (speedup 1.0000x reference)
import functools

import jax
import jax.numpy as jnp
from jax import lax
from jax.experimental import pallas as pl
from jax.experimental.pallas import tpu as pltpu
from jax.experimental.pallas import tpu_sc as plsc

_SEQ = 1500
_AUX = 30
_N2 = 2048
_NVREG = _N2 // 16
_OUTP = 1504


def _cmpx(K, P, i, l, desc):
    ka, kb, pa, pb = K[i], K[l], P[i], P[l]
    swap = (ka < kb) if desc else (ka > kb)
    K[i] = jnp.where(swap, kb, ka)
    K[l] = jnp.where(swap, ka, kb)
    P[i] = jnp.where(swap, pb, pa)
    P[l] = jnp.where(swap, pa, pb)


def _load_group(kref, pref, base, gs):
    K = [kref[pl.ds((base + i) * 16, 16)] for i in range(gs)]
    P = [pref[pl.ds((base + i) * 16, 16)] for i in range(gs)]
    return K, P


def _store_group(kref, pref, base, K, P):
    for i in range(len(K)):
        kref[pl.ds((base + i) * 16, 16)] = K[i]
        pref[pl.ds((base + i) * 16, 16)] = P[i]


def _init_group(kref, pref, tref, g, desc):
    base = g * 4
    K = [kref[pl.ds((base + i) * 16, 16)] for i in range(4)]
    P = [tref[pl.ds((base + i) * 16, 16)] for i in range(4)]
    for i in range(4):
        K[i], P[i] = plsc.sort_key_val(K[i], P[i], descending=(i % 2 == 0))
    _cmpx(K, P, 0, 1, True)
    _cmpx(K, P, 2, 3, False)
    for i in range(4):
        K[i], P[i] = plsc.sort_key_val(K[i], P[i], descending=(i < 2))
    _cmpx(K, P, 0, 2, desc)
    _cmpx(K, P, 1, 3, desc)
    _cmpx(K, P, 0, 1, desc)
    _cmpx(K, P, 2, 3, desc)
    for i in range(4):
        K[i], P[i] = plsc.sort_key_val(K[i], P[i], descending=desc)
    _store_group(kref, pref, base, K, P)


def _tail_group(kref, pref, g, desc):
    base = g * 8
    K, P = _load_group(kref, pref, base, 8)
    for i in range(4):
        _cmpx(K, P, i, i + 4, desc)
    for i in (0, 1, 4, 5):
        _cmpx(K, P, i, i + 2, desc)
    for i in (0, 2, 4, 6):
        _cmpx(K, P, i, i + 1, desc)
    for i in range(8):
        K[i], P[i] = plsc.sort_key_val(K[i], P[i], descending=desc)
    _store_group(kref, pref, base, K, P)


def _tail16_group(kref, pref, g, desc):
    base = g * 16
    K, P = _load_group(kref, pref, base, 16)
    for i in range(8):
        _cmpx(K, P, i, i + 8, desc)
    for h in (0, 8):
        for i in range(4):
            _cmpx(K, P, h + i, h + i + 4, desc)
    for h in (0, 4, 8, 12):
        for i in range(2):
            _cmpx(K, P, h + i, h + i + 2, desc)
    for i in range(0, 16, 2):
        _cmpx(K, P, i, i + 1, desc)
    for i in range(16):
        K[i], P[i] = plsc.sort_key_val(K[i], P[i], descending=desc)
    _store_group(kref, pref, base, K, P)


def _macro_pair(kref, pref, q, desc, j):
    dd = 1 << (j - 4)
    p0 = q * 8
    va = ((p0 >> (j - 4)) << (j - 3)) + (p0 & (dd - 1))
    K1, P1 = _load_group(kref, pref, va, 8)
    K2, P2 = _load_group(kref, pref, va + dd, 8)
    K = K1 + K2
    P = P1 + P2
    for t in range(8):
        _cmpx(K, P, t, t + 8, desc)
    _store_group(kref, pref, va, K[:8], P[:8])
    _store_group(kref, pref, va + dd, K[8:], P[8:])


def _split_loop(n, bb, body):
    if bb >= n:
        def all_body(i, _):
            body(i, True)
            return _
        lax.fori_loop(0, n, all_body, None)
    else:
        def outer(b, _):
            def inner(w, _):
                body(b * 2 * bb + w, True)
                body(b * 2 * bb + bb + w, False)
                return _
            return lax.fori_loop(0, bb, inner, _)
        lax.fori_loop(0, n // (2 * bb), outer, None)


def _sort_row(key_v, pay_v, tmpl_v, gp):
    gpv = gp // 16
    n_init = -(-gpv // 4)
    neg1 = jnp.full((16,), -1.0, jnp.float32)
    padp = jnp.full((16,), (_N2 - 1) << 16, jnp.int32)

    for v in range(gpv, n_init * 4):
        key_v[pl.ds(v * 16, 16)] = neg1

    _split_loop(n_init, 1,
                lambda g, d: _init_group(key_v, pay_v, tmpl_v, g, d))

    def padg_body(v, _):
        key_v[pl.ds(v * 16, 16)] = neg1
        pay_v[pl.ds(v * 16, 16)] = padp
        return _
    lax.fori_loop(n_init * 4, _NVREG, padg_body, None)

    mac = functools.partial(_macro_pair, key_v, pay_v)
    t16 = functools.partial(_tail16_group, key_v, pay_v)

    _split_loop(10, 1, lambda g, d: _tail_group(key_v, pay_v, g, d))
    _tail_group(key_v, pay_v, 10, True)

    _split_loop(6, 1, lambda g, d: t16(g, d))

    _split_loop(4, 2, lambda q, d: mac(q, d, j=8))
    mac(4, True, j=8)
    mac(5, True, j=8)
    _split_loop(4, 2, lambda g, d: t16(g, d))
    t16(4, True)
    t16(5, True)

    for q in range(4):
        mac(q, True, j=9)
    for q in range(4, 8):
        mac(q, False, j=9)
    for q in range(4):
        mac(q, True, j=8)
    mac(6, False, j=8)
    mac(7, False, j=8)
    for g in range(4):
        t16(g, True)
    t16(6, False)
    t16(7, False)

    for q in (5, 6, 7):
        mac(q, True, j=10)
    for q in (0, 1, 2, 3, 5, 6, 7):
        mac(q, True, j=9)
    for q in range(6):
        mac(q, True, j=8)
    _split_loop(6, 8, lambda g, d: t16(g, d))


def _extract(pay_v, out_v):
    nfull = (_OUTP // 16) // 8 * 8

    def ext_body(v8, _):
        for t in range(8):
            s = (v8 * 8 + t) * 16
            out_v[pl.ds(s, 16)] = pay_v[pl.ds(s, 16)] & 0xFFFF
        return _
    lax.fori_loop(0, nfull // 8, ext_body, None)
    for v in range(nfull, _OUTP // 16):
        out_v[pl.ds(v * 16, 16)] = pay_v[pl.ds(v * 16, 16)] & 0xFFFF


def _sc_body(rows_per_w, gp, xw_hbm, tmpl_hbm, out_hbm,
             key_a, pay_a, out_a, key_b, pay_b, out_b, tmpl_v,
             isa, isb, osa, osb):
    nc = 2
    wid = lax.axis_index("s") * nc + lax.axis_index("c")
    base = wid * rows_per_w
    pltpu.sync_copy(tmpl_hbm, tmpl_v)

    def in_start(r, kv, sem):
        pltpu.async_copy(xw_hbm.at[r], kv.at[pl.ds(0, gp)], sem)

    def in_wait(r, kv, sem):
        pltpu.make_async_copy(xw_hbm.at[r], kv.at[pl.ds(0, gp)], sem).wait()

    def out_start(r, ov, sem):
        pltpu.async_copy(ov, out_hbm.at[r], sem)

    def out_wait(r, ov, sem):
        pltpu.make_async_copy(ov, out_hbm.at[r], sem).wait()

    np2 = rows_per_w // 2
    in_start(base, key_a, isa)

    def pair_body(p2, _):
        r0 = base + 2 * p2
        r1 = r0 + 1

        in_wait(r0, key_a, isa)
        in_start(r1, key_b, isb)
        _sort_row(key_a, pay_a, tmpl_v, gp)

        @pl.when(p2 > 0)
        def _():
            out_wait(r0, out_a, osa)
        _extract(pay_a, out_a)
        out_start(r0, out_a, osa)

        in_wait(r1, key_b, isb)

        @pl.when(p2 + 1 < np2)
        def _():
            in_start(r0 + 2, key_a, isa)
        _sort_row(key_b, pay_b, tmpl_v, gp)

        @pl.when(p2 > 0)
        def _():
            out_wait(r1, out_b, osb)
        _extract(pay_b, out_b)
        out_start(r1, out_b, osb)
        return _

    lax.fori_loop(0, np2, pair_body, None)
    out_wait(base, out_a, osa)
    out_wait(base, out_b, osb)


def kernel(X, gene_mask, token_ids, technology_mean):
    N, G = X.shape
    gp = ((G + 15) // 16) * 16

    tech = jnp.nan_to_num(technology_mean)
    tech = tech + (tech == 0).astype(jnp.float32)
    tech_g = jnp.take(tech, token_ids)
    w = jnp.where(gene_mask, 1.0 / tech_g, 0.0)
    xw = X * w[None, :]
    xw_p = jnp.concatenate(
        [xw, jnp.full((N, gp - G), -1.0, jnp.float32)], axis=1)

    idx = jnp.arange(G, dtype=jnp.int32)
    tmpl_g = (idx << 16) | (token_ids.astype(jnp.int32) + _AUX)
    tmpl = jnp.concatenate(
        [tmpl_g, jnp.full((_N2 - G,), (_N2 - 1) << 16, jnp.int32)])

    mesh = plsc.VectorSubcoreMesh(core_axis_name="c", subcore_axis_name="s")
    nw = 32
    rows_per_w = N // nw

    sc = functools.partial(
        pl.kernel,
        out_type=jax.ShapeDtypeStruct((N, _OUTP), jnp.int32),
        mesh=mesh,
        compiler_params=pltpu.CompilerParams(
            needs_layout_passes=False, use_tc_tiling_on_sc=False),
        scratch_types=[
            pltpu.VMEM((_N2,), jnp.float32),
            pltpu.VMEM((_N2,), jnp.int32),
            pltpu.VMEM((_OUTP,), jnp.int32),
            pltpu.VMEM((_N2,), jnp.float32),
            pltpu.VMEM((_N2,), jnp.int32),
            pltpu.VMEM((_OUTP,), jnp.int32),
            pltpu.VMEM((_N2,), jnp.int32),
            pltpu.SemaphoreType.DMA,
            pltpu.SemaphoreType.DMA,
            pltpu.SemaphoreType.DMA,
            pltpu.SemaphoreType.DMA,
        ],
    )(functools.partial(_sc_body, rows_per_w, gp))

    out_p = sc(xw_p, tmpl)
    return out_p[:, :_SEQ]


# speedup vs baseline: 1.2745x; 1.2745x over previous
import functools

import jax
import jax.numpy as jnp
from jax import lax
from jax.experimental import pallas as pl
from jax.experimental.pallas import tpu as pltpu
from jax.experimental.pallas import tpu_sc as plsc

_SEQ = 1500
_AUX = 30
_N2 = 2048
_NVREG = _N2 // 16
_OUTP = 1504


def _cmpx(K, P, i, l, desc):
    ka, kb, pa, pb = K[i], K[l], P[i], P[l]
    swap = (ka < kb) if desc else (ka > kb)
    K[i] = jnp.where(swap, kb, ka)
    K[l] = jnp.where(swap, ka, kb)
    P[i] = jnp.where(swap, pb, pa)
    P[l] = jnp.where(swap, pa, pb)


def _load_group(kref, pref, base, gs):
    K = [kref[pl.ds((base + i) * 16, 16)] for i in range(gs)]
    P = [pref[pl.ds((base + i) * 16, 16)] for i in range(gs)]
    return K, P


def _store_group(kref, pref, base, K, P):
    for i in range(len(K)):
        kref[pl.ds((base + i) * 16, 16)] = K[i]
        pref[pl.ds((base + i) * 16, 16)] = P[i]


def _init_group(kref, pref, tref, g, desc):
    base = g * 4
    K = [kref[pl.ds((base + i) * 16, 16)] for i in range(4)]
    P = [tref[pl.ds((base + i) * 16, 16)] for i in range(4)]
    for i in range(4):
        K[i], P[i] = plsc.sort_key_val(K[i], P[i], descending=(i % 2 == 0))
    _cmpx(K, P, 0, 1, True)
    _cmpx(K, P, 2, 3, False)
    for i in range(4):
        K[i], P[i] = plsc.sort_key_val(K[i], P[i], descending=(i < 2))
    _cmpx(K, P, 0, 2, desc)
    _cmpx(K, P, 1, 3, desc)
    _cmpx(K, P, 0, 1, desc)
    _cmpx(K, P, 2, 3, desc)
    for i in range(4):
        K[i], P[i] = plsc.sort_key_val(K[i], P[i], descending=desc)
    _store_group(kref, pref, base, K, P)


def _tail_group(kref, pref, g, desc):
    base = g * 8
    K, P = _load_group(kref, pref, base, 8)
    for i in range(4):
        _cmpx(K, P, i, i + 4, desc)
    for i in (0, 1, 4, 5):
        _cmpx(K, P, i, i + 2, desc)
    for i in (0, 2, 4, 6):
        _cmpx(K, P, i, i + 1, desc)
    for i in range(8):
        K[i], P[i] = plsc.sort_key_val(K[i], P[i], descending=desc)
    _store_group(kref, pref, base, K, P)


def _tail16_group(kref, pref, g, desc):
    base = g * 16
    K, P = _load_group(kref, pref, base, 16)
    for i in range(8):
        _cmpx(K, P, i, i + 8, desc)
    for h in (0, 8):
        for i in range(4):
            _cmpx(K, P, h + i, h + i + 4, desc)
    for h in (0, 4, 8, 12):
        for i in range(2):
            _cmpx(K, P, h + i, h + i + 2, desc)
    for i in range(0, 16, 2):
        _cmpx(K, P, i, i + 1, desc)
    for i in range(16):
        K[i], P[i] = plsc.sort_key_val(K[i], P[i], descending=desc)
    _store_group(kref, pref, base, K, P)


def _macro_pair(kref, pref, q, desc, j):
    dd = 1 << (j - 4)
    p0 = q * 8
    va = ((p0 >> (j - 4)) << (j - 3)) + (p0 & (dd - 1))
    K1, P1 = _load_group(kref, pref, va, 8)
    K2, P2 = _load_group(kref, pref, va + dd, 8)
    K = K1 + K2
    P = P1 + P2
    for t in range(8):
        _cmpx(K, P, t, t + 8, desc)
    _store_group(kref, pref, va, K[:8], P[:8])
    _store_group(kref, pref, va + dd, K[8:], P[8:])


def _split_loop(n, bb, body):
    if bb >= n:
        def all_body(i, _):
            body(i, True)
            return _
        lax.fori_loop(0, n, all_body, None)
    else:
        def outer(b, _):
            def inner(w, _):
                body(b * 2 * bb + w, True)
                body(b * 2 * bb + bb + w, False)
                return _
            return lax.fori_loop(0, bb, inner, _)
        lax.fori_loop(0, n // (2 * bb), outer, None)


def _sort_row(key_v, pay_v, tmpl_v, gp):
    gpv = gp // 16
    n_init = -(-gpv // 4)
    neg1 = jnp.full((16,), -1.0, jnp.float32)
    padp = jnp.full((16,), (_N2 - 1) << 16, jnp.int32)

    for v in range(gpv, n_init * 4):
        key_v[pl.ds(v * 16, 16)] = neg1

    _split_loop(n_init, 1,
                lambda g, d: _init_group(key_v, pay_v, tmpl_v, g, d))

    def padg_body(v, _):
        key_v[pl.ds(v * 16, 16)] = neg1
        pay_v[pl.ds(v * 16, 16)] = padp
        return _
    lax.fori_loop(n_init * 4, _NVREG, padg_body, None)

    for k in range(7, 12):
        if k == 7:
            _split_loop(10, 1,
                        lambda g, d: _tail_group(key_v, pay_v, g, d))
            _tail_group(key_v, pay_v, 10, True)
            continue
        for j in range(k - 1, 7, -1):
            _split_loop(_NVREG // 16, 1 << (k - 8),
                        functools.partial(_macro_pair, key_v, pay_v, j=j))
        _split_loop(6 if k == 8 else _NVREG // 16, 1 << (k - 8),
                    lambda g, d: _tail16_group(key_v, pay_v, g, d))


def _extract(pay_v, out_v):
    nfull = (_OUTP // 16) // 8 * 8

    def ext_body(v8, _):
        for t in range(8):
            s = (v8 * 8 + t) * 16
            out_v[pl.ds(s, 16)] = pay_v[pl.ds(s, 16)] & 0xFFFF
        return _
    lax.fori_loop(0, nfull // 8, ext_body, None)
    for v in range(nfull, _OUTP // 16):
        out_v[pl.ds(v * 16, 16)] = pay_v[pl.ds(v * 16, 16)] & 0xFFFF


def _sc_body(rows_per_w, gp, xw_hbm, tmpl_hbm, out_hbm,
             key_a, pay_a, out_a, key_b, pay_b, out_b, tmpl_v,
             isa, isb, osa, osb):
    nc = 2
    wid = lax.axis_index("s") * nc + lax.axis_index("c")
    base = wid * rows_per_w
    pltpu.sync_copy(tmpl_hbm, tmpl_v)

    def in_start(r, kv, sem):
        pltpu.async_copy(xw_hbm.at[r], kv.at[pl.ds(0, gp)], sem)

    def in_wait(r, kv, sem):
        pltpu.make_async_copy(xw_hbm.at[r], kv.at[pl.ds(0, gp)], sem).wait()

    def out_start(r, ov, sem):
        pltpu.async_copy(ov, out_hbm.at[r], sem)

    def out_wait(r, ov, sem):
        pltpu.make_async_copy(ov, out_hbm.at[r], sem).wait()

    np2 = rows_per_w // 2
    in_start(base, key_a, isa)

    def pair_body(p2, _):
        r0 = base + 2 * p2
        r1 = r0 + 1

        in_wait(r0, key_a, isa)
        in_start(r1, key_b, isb)
        _sort_row(key_a, pay_a, tmpl_v, gp)

        @pl.when(p2 > 0)
        def _():
            out_wait(r0, out_a, osa)
        _extract(pay_a, out_a)
        out_start(r0, out_a, osa)

        in_wait(r1, key_b, isb)

        @pl.when(p2 + 1 < np2)
        def _():
            in_start(r0 + 2, key_a, isa)
        _sort_row(key_b, pay_b, tmpl_v, gp)

        @pl.when(p2 > 0)
        def _():
            out_wait(r1, out_b, osb)
        _extract(pay_b, out_b)
        out_start(r1, out_b, osb)
        return _

    lax.fori_loop(0, np2, pair_body, None)
    out_wait(base, out_a, osa)
    out_wait(base, out_b, osb)


def kernel(X, gene_mask, token_ids, technology_mean):
    N, G = X.shape
    gp = ((G + 15) // 16) * 16

    tech = jnp.nan_to_num(technology_mean)
    tech = tech + (tech == 0).astype(jnp.float32)
    tech_g = jnp.take(tech, token_ids)
    w = jnp.where(gene_mask, 1.0 / tech_g, 0.0)
    xw = X * w[None, :]
    xw_p = jnp.concatenate(
        [xw, jnp.full((N, gp - G), -1.0, jnp.float32)], axis=1)

    idx = jnp.arange(G, dtype=jnp.int32)
    tmpl_g = (idx << 16) | (token_ids.astype(jnp.int32) + _AUX)
    tmpl = jnp.concatenate(
        [tmpl_g, jnp.full((_N2 - G,), (_N2 - 1) << 16, jnp.int32)])

    mesh = plsc.VectorSubcoreMesh(core_axis_name="c", subcore_axis_name="s")
    nw = 32
    rows_per_w = N // nw

    sc = functools.partial(
        pl.kernel,
        out_type=jax.ShapeDtypeStruct((N, _OUTP), jnp.int32),
        mesh=mesh,
        compiler_params=pltpu.CompilerParams(
            needs_layout_passes=False, use_tc_tiling_on_sc=False),
        scratch_types=[
            pltpu.VMEM((_N2,), jnp.float32),
            pltpu.VMEM((_N2,), jnp.int32),
            pltpu.VMEM((_OUTP,), jnp.int32),
            pltpu.VMEM((_N2,), jnp.float32),
            pltpu.VMEM((_N2,), jnp.int32),
            pltpu.VMEM((_OUTP,), jnp.int32),
            pltpu.VMEM((_N2,), jnp.int32),
            pltpu.SemaphoreType.DMA,
            pltpu.SemaphoreType.DMA,
            pltpu.SemaphoreType.DMA,
            pltpu.SemaphoreType.DMA,
        ],
    )(functools.partial(_sc_body, rows_per_w, gp))

    out_p = sc(xw_p, tmpl)
    return out_p[:, :_SEQ]
